# baseline (speedup 1.0000x reference)
import numpy as np
import jax
import jax.numpy as jnp
from jax import lax
from jax.experimental import pallas as pl
from jax.experimental.pallas import tpu as pltpu

_EMB_DIM = 1024
_HALF = _EMB_DIM // 2
_D_ROWS = 512
_N_BUF = 4


def _inv_freq_row():
    scale = np.float32(np.log(10000.0) / (_HALF - 1))
    j = jax.lax.broadcasted_iota(jnp.int32, (1, _HALF), 1).astype(jnp.float32)
    return jnp.exp(j * (-scale))


def _gen_body(out_hbm, sin_d, cos_d, sin_b, cos_b, bufs, sems):
    n_bases = sin_b.shape[0]
    seq_len = _D_ROWS * n_bases // 2
    inv_freq = _inv_freq_row()

    d = jax.lax.broadcasted_iota(jnp.int32, (_D_ROWS, 1), 0).astype(jnp.float32)
    angle_d = d * inv_freq
    sin_d[...] = jnp.sin(angle_d)
    cos_d[...] = jnp.cos(angle_d)
    b = jax.lax.broadcasted_iota(jnp.int32, (n_bases, 1), 0) * _D_ROWS
    angle_b = (b - seq_len).astype(jnp.float32) * inv_freq
    sin_b[...] = jnp.sin(angle_b)
    cos_b[...] = jnp.cos(angle_b)

    sd = sin_d[...]
    cd = cos_d[...]

    def chunk_copy(bidx, buf):
        return pltpu.make_async_copy(
            bufs.at[buf], out_hbm.at[pl.ds(bidx * _D_ROWS, _D_ROWS), :],
            sems.at[buf],
        )

    def compute_chunk(bidx, buf):
        s0 = sin_b[pl.ds(bidx, 1), :]
        c0 = cos_b[pl.ds(bidx, 1), :]
        bufs[buf, :, :_HALF] = s0 * cd + c0 * sd
        bufs[buf, :, _HALF:] = c0 * cd - s0 * sd
        chunk_copy(bidx, buf).start()

    for buf in range(_N_BUF):
        compute_chunk(buf, buf)

    def outer(g, _):
        for buf in range(_N_BUF):
            bidx = g * _N_BUF + buf
            chunk_copy(bidx - _N_BUF, buf).wait()
            compute_chunk(bidx, buf)
        return _

    n_outer = n_bases // _N_BUF
    lax.fori_loop(1, n_outer, outer, 0)

    for buf in range(_N_BUF):
        chunk_copy(n_bases - _N_BUF + buf, buf).wait()


def kernel(input, emb_table):
    seq_len = input.shape[1]
    rows = 2 * seq_len
    return pl.pallas_call(
        _gen_body,
        out_shape=jax.ShapeDtypeStruct((rows, _EMB_DIM), jnp.float32),
        out_specs=pl.BlockSpec(memory_space=pltpu.HBM),
        scratch_shapes=[
            pltpu.VMEM((_D_ROWS, _HALF), jnp.float32),
            pltpu.VMEM((_D_ROWS, _HALF), jnp.float32),
            pltpu.VMEM((rows // _D_ROWS, _HALF), jnp.float32),
            pltpu.VMEM((rows // _D_ROWS, _HALF), jnp.float32),
            pltpu.VMEM((_N_BUF, _D_ROWS, _EMB_DIM), jnp.float32),
            pltpu.SemaphoreType.DMA((_N_BUF,)),
        ],
    )()


# speedup vs baseline: 1.1851x; 1.1851x over previous
import numpy as np
import jax
import jax.numpy as jnp
from jax import lax
from jax.experimental import pallas as pl
from jax.experimental.pallas import tpu as pltpu

_EMB_DIM = 1024
_HALF = _EMB_DIM // 2
_D_ROWS = 256
_N_BUF = 4


def _inv_freq_row():
    scale = np.float32(np.log(10000.0) / (_HALF - 1))
    j = jax.lax.broadcasted_iota(jnp.int32, (1, _HALF), 1).astype(jnp.float32)
    return jnp.exp(j * (-scale))


def _gen_body(out_hbm, sin_d, cos_d, sin_b, cos_b, s16s_v, c16s_v, bufs, sems):
    n_bases = sin_b.shape[0]
    seq_len = _D_ROWS * n_bases // 2
    inv_freq = _inv_freq_row()
    n_fine = _D_ROWS // 16

    d16 = jax.lax.broadcasted_iota(jnp.int32, (n_fine, 1), 0).astype(jnp.float32)
    a_fine = d16 * inv_freq
    s16 = jnp.sin(a_fine)
    c16 = jnp.cos(a_fine)
    a_coarse = d16 * (inv_freq * np.float32(n_fine))
    s16s_v[...] = jnp.sin(a_coarse)
    c16s_v[...] = jnp.cos(a_coarse)
    for a in range(_D_ROWS // n_fine):
        sa = s16s_v[pl.ds(a, 1), :]
        ca = c16s_v[pl.ds(a, 1), :]
        rows = pl.ds(a * n_fine, n_fine)
        sin_d[rows, :] = sa * c16 + ca * s16
        cos_d[rows, :] = ca * c16 - sa * s16

    d64 = jax.lax.broadcasted_iota(jnp.int32, (n_bases, 1), 0).astype(jnp.float32)
    a_256 = d64 * (inv_freq * np.float32(_D_ROWS))
    s256 = jnp.sin(a_256)
    c256 = jnp.cos(a_256)
    a_shift = inv_freq * np.float32(-seq_len)
    s_s = jnp.sin(a_shift)
    c_s = jnp.cos(a_shift)
    sin_b[...] = s256 * c_s + c256 * s_s
    cos_b[...] = c256 * c_s - s256 * s_s

    sd = sin_d[...]
    cd = cos_d[...]

    def chunk_copy(bidx, buf):
        return pltpu.make_async_copy(
            bufs.at[buf], out_hbm.at[pl.ds(bidx * _D_ROWS, _D_ROWS), :],
            sems.at[buf],
        )

    def compute_chunk(bidx, buf):
        s0 = sin_b[pl.ds(bidx, 1), :]
        c0 = cos_b[pl.ds(bidx, 1), :]
        bufs[buf, :, :_HALF] = s0 * cd + c0 * sd
        bufs[buf, :, _HALF:] = c0 * cd - s0 * sd
        chunk_copy(bidx, buf).start()

    for buf in range(_N_BUF):
        compute_chunk(buf, buf)

    def outer(g, _):
        for buf in range(_N_BUF):
            bidx = g * _N_BUF + buf
            chunk_copy(bidx - _N_BUF, buf).wait()
            compute_chunk(bidx, buf)
        return _

    n_outer = n_bases // _N_BUF
    lax.fori_loop(1, n_outer, outer, 0)

    for buf in range(_N_BUF):
        chunk_copy(n_bases - _N_BUF + buf, buf).wait()


def kernel(input, emb_table):
    seq_len = input.shape[1]
    rows = 2 * seq_len
    return pl.pallas_call(
        _gen_body,
        out_shape=jax.ShapeDtypeStruct((rows, _EMB_DIM), jnp.float32),
        out_specs=pl.BlockSpec(memory_space=pltpu.HBM),
        scratch_shapes=[
            pltpu.VMEM((_D_ROWS, _HALF), jnp.float32),
            pltpu.VMEM((_D_ROWS, _HALF), jnp.float32),
            pltpu.VMEM((rows // _D_ROWS, _HALF), jnp.float32),
            pltpu.VMEM((rows // _D_ROWS, _HALF), jnp.float32),
            pltpu.VMEM((_D_ROWS // 16, _HALF), jnp.float32),
            pltpu.VMEM((_D_ROWS // 16, _HALF), jnp.float32),
            pltpu.VMEM((_N_BUF, _D_ROWS, _EMB_DIM), jnp.float32),
            pltpu.SemaphoreType.DMA((_N_BUF,)),
        ],
    )()
